# baseline (speedup 1.0000x reference)
import functools

import jax
import jax.numpy as jnp
from jax import lax
from jax.experimental import pallas as pl
from jax.experimental.pallas import tpu as pltpu
from jax.experimental.pallas import tpu_sc as plsc

EMB_DIM = 256
PROJ_DIM = 128
NATURE_BINS = 32
NATURE_DIM = 32
BATCH = 16384

_NHALF = 2
_HB = BATCH // _NHALF

_NC = 2
_NS = 16
_NW = _NC * _NS
_BPW = _HB // _NW
_CHUNK = 128
_NCHUNK = _BPW // _CHUNK


def _sc_gather_body(code_ids_h, name_ids_h, code_tab_h, name_tab_h,
                    code_out_h, name_out_h,
                    idx_v, rows_a, rows_b, sem_a, sem_b):
    wid = lax.axis_index("s") * _NC + lax.axis_index("c")
    base = wid * _BPW

    for ids_h, tab_h, out_h in ((code_ids_h, code_tab_h, code_out_h),
                                (name_ids_h, name_tab_h, name_out_h)):
        pltpu.sync_copy(ids_h.at[wid], idx_v)
        bufs = (rows_a, rows_b)
        sems = (sem_a, sem_b)
        pending = [None, None]
        pending[0] = pltpu.async_copy(tab_h.at[idx_v.at[0]], bufs[0], sems[0])
        for j in range(_NCHUNK):
            if j + 1 < _NCHUNK:
                pending[(j + 1) % 2] = pltpu.async_copy(
                    tab_h.at[idx_v.at[j + 1]], bufs[(j + 1) % 2],
                    sems[(j + 1) % 2])
            pending[j % 2].wait()
            pltpu.sync_copy(bufs[j % 2],
                            out_h.at[pl.ds(base + j * _CHUNK, _CHUNK)])


_sc_gather = functools.partial(
    pl.kernel,
    out_type=(jax.ShapeDtypeStruct((_HB, PROJ_DIM), jnp.float32),
              jax.ShapeDtypeStruct((_HB, PROJ_DIM), jnp.float32)),
    mesh=plsc.VectorSubcoreMesh(core_axis_name="c", subcore_axis_name="s"),
    scratch_types=[
        pltpu.VMEM((_NCHUNK, _CHUNK), jnp.int32),
        pltpu.VMEM((_CHUNK, PROJ_DIM), jnp.float32),
        pltpu.VMEM((_CHUNK, PROJ_DIM), jnp.float32),
        pltpu.SemaphoreType.DMA,
        pltpu.SemaphoreType.DMA,
    ],
)(_sc_gather_body)


_BLK = 4096
_GRID_H = _HB // _BLK


def _tc_compute(nid_ref, cv_ref, nv_ref, ntab_ref, w_ref, b_ref, g_ref,
                be_ref, out_ref):
    cv = cv_ref[...].astype(jnp.bfloat16)
    nv = nv_ref[...].astype(jnp.bfloat16)
    nid = nid_ref[0, 0, :]
    w = w_ref[...].astype(jnp.bfloat16)
    onehot = (nid[:, None]
              == lax.broadcasted_iota(jnp.int32, (_BLK, NATURE_DIM), 1)
              ).astype(jnp.bfloat16)
    nat_w = jnp.dot(ntab_ref[...].astype(jnp.bfloat16), w[2 * PROJ_DIM:, :],
                    preferred_element_type=jnp.float32
                    ).astype(jnp.bfloat16)
    y = (jnp.dot(cv, w[:PROJ_DIM, :], preferred_element_type=jnp.float32)
         + jnp.dot(nv, w[PROJ_DIM:2 * PROJ_DIM, :],
                   preferred_element_type=jnp.float32)
         + jnp.dot(onehot, nat_w, preferred_element_type=jnp.float32)
         + b_ref[...])
    mean = jnp.mean(y, axis=-1, keepdims=True)
    var = jnp.mean((y - mean) ** 2, axis=-1, keepdims=True)
    out_ref[...] = ((y - mean) * lax.rsqrt(var + 1e-3) * g_ref[...]
                    + be_ref[...])


def _tc_body_first(nid_ref, cv_ref, nv_ref, ntab_ref, w_ref, b_ref, g_ref,
                   be_ref, out_ref):
    _tc_compute(nid_ref, cv_ref, nv_ref, ntab_ref, w_ref, b_ref, g_ref,
                be_ref, out_ref)


def _tc_body_second(prev_ref, nid_ref, cv_ref, nv_ref, ntab_ref, w_ref,
                    b_ref, g_ref, be_ref, out_ref):
    del prev_ref
    _tc_compute(nid_ref, cv_ref, nv_ref, ntab_ref, w_ref, b_ref, g_ref,
                be_ref, out_ref)


_COMMON_IN_SPECS = [
    pl.BlockSpec((1, 1, _BLK), lambda i: (i, 0, 0)),
    pl.BlockSpec((_BLK, PROJ_DIM), lambda i: (i, 0)),
    pl.BlockSpec((_BLK, PROJ_DIM), lambda i: (i, 0)),
    pl.BlockSpec((NATURE_BINS, NATURE_DIM), lambda i: (0, 0)),
    pl.BlockSpec((2 * PROJ_DIM + NATURE_DIM, EMB_DIM), lambda i: (0, 0)),
    pl.BlockSpec((1, EMB_DIM), lambda i: (0, 0)),
    pl.BlockSpec((1, EMB_DIM), lambda i: (0, 0)),
    pl.BlockSpec((1, EMB_DIM), lambda i: (0, 0)),
]

_tc_first = pl.pallas_call(
    _tc_body_first,
    grid=(_GRID_H,),
    in_specs=_COMMON_IN_SPECS,
    out_specs=pl.BlockSpec((_BLK, EMB_DIM), lambda i: (i, 0)),
    out_shape=jax.ShapeDtypeStruct((BATCH, EMB_DIM), jnp.float32),
)

_tc_second = pl.pallas_call(
    _tc_body_second,
    grid=(_GRID_H,),
    in_specs=[pl.BlockSpec(memory_space=pl.ANY)] + _COMMON_IN_SPECS,
    out_specs=pl.BlockSpec((_BLK, EMB_DIM), lambda i: (i + _GRID_H, 0)),
    out_shape=jax.ShapeDtypeStruct((BATCH, EMB_DIM), jnp.float32),
    input_output_aliases={0: 0},
)


def kernel(code_ids, name_ids, nature_ids, code_table, name_table,
           nature_table, W, b, gamma, beta):
    ci = code_ids.astype(jnp.int32).reshape(_NHALF, _NW, _NCHUNK, _CHUNK)
    ni = name_ids.astype(jnp.int32).reshape(_NHALF, _NW, _NCHUNK, _CHUNK)
    ti = nature_ids.astype(jnp.int32).reshape(_NHALF, _GRID_H, 1, _BLK)
    b2 = b.reshape(1, EMB_DIM)
    g2 = gamma.reshape(1, EMB_DIM)
    be2 = beta.reshape(1, EMB_DIM)

    cv0, nv0 = _sc_gather(ci[0], ni[0], code_table, name_table)
    cv1, nv1 = _sc_gather(ci[1], ni[1], code_table, name_table)
    y0 = _tc_first(ti[0], cv0, nv0, nature_table, W, b2, g2, be2)
    return _tc_second(y0, ti[1], cv1, nv1, nature_table, W, b2, g2, be2)


# speedup vs baseline: 1.0611x; 1.0611x over previous
import functools

import jax
import jax.numpy as jnp
from jax import lax
from jax.experimental import pallas as pl
from jax.experimental.pallas import tpu as pltpu
from jax.experimental.pallas import tpu_sc as plsc

EMB_DIM = 256
PROJ_DIM = 128
NATURE_BINS = 32
NATURE_DIM = 32
BATCH = 16384

_NHALF = 2
_HB = BATCH // _NHALF

_NC = 2
_NS = 16
_NW = _NC * _NS
_BPW = _HB // _NW
_CHUNK = 128
_NCHUNK = _BPW // _CHUNK


def _sc_gather_body(half, code_ids_h, name_ids_h, code_tab_h, name_tab_h,
                    code_out_h, name_out_h,
                    idx_c, idx_n, g0, g1, g2, g3,
                    sg0, sg1, sg2, sg3, ss0, ss1, ss2, ss3):
    wid = lax.axis_index("s") * _NC + lax.axis_index("c")
    base = wid * _BPW
    gbufs = (g0, g1, g2, g3)
    gsems = (sg0, sg1, sg2, sg3)
    ssems = (ss0, ss1, ss2, ss3)

    pltpu.sync_copy(code_ids_h.at[half, wid], idx_c)
    pltpu.sync_copy(name_ids_h.at[half, wid], idx_n)
    pend = []
    for j in range(_NCHUNK):
        pend.append(pltpu.async_copy(code_tab_h.at[idx_c.at[j]],
                                     gbufs[j], gsems[j]))
    for j in range(_NCHUNK):
        pend.append(pltpu.async_copy(name_tab_h.at[idx_n.at[j]],
                                     gbufs[_NCHUNK + j], gsems[_NCHUNK + j]))
    stores = []
    for j in range(_NCHUNK):
        pend[j].wait()
        stores.append(pltpu.async_copy(
            gbufs[j], code_out_h.at[pl.ds(base + j * _CHUNK, _CHUNK)],
            ssems[j]))
    for j in range(_NCHUNK):
        pend[_NCHUNK + j].wait()
        stores.append(pltpu.async_copy(
            gbufs[_NCHUNK + j],
            name_out_h.at[pl.ds(base + j * _CHUNK, _CHUNK)],
            ssems[_NCHUNK + j]))
    for s in stores:
        s.wait()


def _make_sc_gather(half):
    return functools.partial(
        pl.kernel,
        out_type=(jax.ShapeDtypeStruct((_HB, PROJ_DIM), jnp.float32),
                  jax.ShapeDtypeStruct((_HB, PROJ_DIM), jnp.float32)),
        mesh=plsc.VectorSubcoreMesh(core_axis_name="c", subcore_axis_name="s"),
        scratch_types=[
            pltpu.VMEM((_NCHUNK, _CHUNK), jnp.int32),
            pltpu.VMEM((_NCHUNK, _CHUNK), jnp.int32),
            pltpu.VMEM((_CHUNK, PROJ_DIM), jnp.float32),
            pltpu.VMEM((_CHUNK, PROJ_DIM), jnp.float32),
            pltpu.VMEM((_CHUNK, PROJ_DIM), jnp.float32),
            pltpu.VMEM((_CHUNK, PROJ_DIM), jnp.float32),
        ] + [pltpu.SemaphoreType.DMA] * 8,
    )(functools.partial(_sc_gather_body, half))


_sc_gather_0 = _make_sc_gather(0)
_sc_gather_1 = _make_sc_gather(1)


_BLK = 2048
_GRID_H = _HB // _BLK


def _tc_compute(nid_ref, cv_ref, nv_ref, ntab_ref, w_ref, b_ref, g_ref,
                be_ref, out_ref):
    cv = cv_ref[...].astype(jnp.bfloat16)
    nv = nv_ref[...].astype(jnp.bfloat16)
    nid = nid_ref[0, 0, :]
    w = w_ref[...].astype(jnp.bfloat16)
    onehot = (nid[:, None]
              == lax.broadcasted_iota(jnp.int32, (_BLK, NATURE_DIM), 1)
              ).astype(jnp.bfloat16)
    nat_w = jnp.dot(ntab_ref[...].astype(jnp.bfloat16), w[2 * PROJ_DIM:, :],
                    preferred_element_type=jnp.float32
                    ).astype(jnp.bfloat16)
    y = (jnp.dot(cv, w[:PROJ_DIM, :], preferred_element_type=jnp.float32)
         + jnp.dot(nv, w[PROJ_DIM:2 * PROJ_DIM, :],
                   preferred_element_type=jnp.float32)
         + jnp.dot(onehot, nat_w, preferred_element_type=jnp.float32)
         + b_ref[...])
    mean = jnp.mean(y, axis=-1, keepdims=True)
    var = jnp.mean((y - mean) ** 2, axis=-1, keepdims=True)
    out_ref[...] = ((y - mean) * lax.rsqrt(var + 1e-3) * g_ref[...]
                    + be_ref[...])


def _tc_body_first(nid_ref, cv_ref, nv_ref, ntab_ref, w_ref, b_ref, g_ref,
                   be_ref, out_ref):
    _tc_compute(nid_ref, cv_ref, nv_ref, ntab_ref, w_ref, b_ref, g_ref,
                be_ref, out_ref)


def _tc_body_second(prev_ref, nid_ref, cv_ref, nv_ref, ntab_ref, w_ref,
                    b_ref, g_ref, be_ref, out_ref):
    del prev_ref
    _tc_compute(nid_ref, cv_ref, nv_ref, ntab_ref, w_ref, b_ref, g_ref,
                be_ref, out_ref)


def _common_in_specs(half):
    return [
        pl.BlockSpec((1, 1, _BLK), lambda i: (i + half * _GRID_H, 0, 0)),
        pl.BlockSpec((_BLK, PROJ_DIM), lambda i: (i, 0)),
        pl.BlockSpec((_BLK, PROJ_DIM), lambda i: (i, 0)),
        pl.BlockSpec((NATURE_BINS, NATURE_DIM), lambda i: (0, 0)),
        pl.BlockSpec((2 * PROJ_DIM + NATURE_DIM, EMB_DIM), lambda i: (0, 0)),
        pl.BlockSpec((1, EMB_DIM), lambda i: (0, 0)),
        pl.BlockSpec((1, EMB_DIM), lambda i: (0, 0)),
        pl.BlockSpec((1, EMB_DIM), lambda i: (0, 0)),
    ]


_tc_first = pl.pallas_call(
    _tc_body_first,
    grid=(_GRID_H,),
    in_specs=_common_in_specs(0),
    out_specs=pl.BlockSpec((_BLK, EMB_DIM), lambda i: (i, 0)),
    out_shape=jax.ShapeDtypeStruct((BATCH, EMB_DIM), jnp.float32),
)

_tc_second = pl.pallas_call(
    _tc_body_second,
    grid=(_GRID_H,),
    in_specs=[pl.BlockSpec(memory_space=pl.ANY)] + _common_in_specs(1),
    out_specs=pl.BlockSpec((_BLK, EMB_DIM), lambda i: (i + _GRID_H, 0)),
    out_shape=jax.ShapeDtypeStruct((BATCH, EMB_DIM), jnp.float32),
    input_output_aliases={0: 0},
)


def kernel(code_ids, name_ids, nature_ids, code_table, name_table,
           nature_table, W, b, gamma, beta):
    ci = code_ids.astype(jnp.int32).reshape(_NHALF, _NW, _NCHUNK, _CHUNK)
    ni = name_ids.astype(jnp.int32).reshape(_NHALF, _NW, _NCHUNK, _CHUNK)
    ti = nature_ids.astype(jnp.int32).reshape(_NHALF * _GRID_H, 1, _BLK)
    b2 = b.reshape(1, EMB_DIM)
    g2 = gamma.reshape(1, EMB_DIM)
    be2 = beta.reshape(1, EMB_DIM)

    cv0, nv0 = _sc_gather_0(ci, ni, code_table, name_table)
    cv1, nv1 = _sc_gather_1(ci, ni, code_table, name_table)
    y0 = _tc_first(ti, cv0, nv0, nature_table, W, b2, g2, be2)
    return _tc_second(y0, ti, cv1, nv1, nature_table, W, b2, g2, be2)
